# baseline (device time: 63645 ns/iter reference)
import jax
import jax.numpy as jnp
from jax import lax
from jax.experimental import pallas as pl
from jax.experimental.pallas import tpu as pltpu

N_GLOBAL = 4096
EPS = 1e-5
BM = 1024


def _stats_kernel(x):
    m_per, n_per = x.shape
    n_blocks = m_per // BM

    def body(x_ref, o_ref, acc, recv, send_sem, recv_sem):
        i = pl.program_id(0)
        my_x = lax.axis_index("x")
        my_y = lax.axis_index("y")

        xb = x_ref[:, :]
        ones = jnp.ones((n_per, 1), jnp.float32)
        s1 = jax.lax.dot_general(
            xb, ones, (((1,), (0,)), ((), ())),
            preferred_element_type=jnp.float32,
        )
        s2 = jax.lax.dot_general(
            xb * xb, ones, (((1,), (0,)), ((), ())),
            preferred_element_type=jnp.float32,
        )
        acc[0, pl.ds(i * BM, BM)] = s1[:, 0]
        acc[1, pl.ds(i * BM, BM)] = s2[:, 0]

        @pl.when(i == n_blocks - 1)
        def _():
            rdma = pltpu.make_async_remote_copy(
                src_ref=acc,
                dst_ref=recv,
                send_sem=send_sem,
                recv_sem=recv_sem,
                device_id=(my_x, 1 - my_y),
                device_id_type=pl.DeviceIdType.MESH,
            )
            rdma.start()
            rdma.wait()
            tot_s1 = acc[0, :] + recv[0, :]
            tot_s2 = acc[1, :] + recv[1, :]
            mean = tot_s1 / N_GLOBAL
            var = tot_s2 / N_GLOBAL - mean * mean
            o_ref[0, :] = mean
            o_ref[1, :] = lax.rsqrt(var + EPS)

    return pl.pallas_call(
        body,
        grid=(n_blocks,),
        in_specs=[pl.BlockSpec((BM, n_per), lambda i: (i, 0))],
        out_specs=pl.BlockSpec((2, m_per), lambda i: (0, 0)),
        out_shape=jax.ShapeDtypeStruct((2, m_per), jnp.float32),
        scratch_shapes=[
            pltpu.VMEM((2, m_per), jnp.float32),
            pltpu.VMEM((2, m_per), jnp.float32),
            pltpu.SemaphoreType.DMA,
            pltpu.SemaphoreType.DMA,
        ],
        compiler_params=pltpu.CompilerParams(
            dimension_semantics=("arbitrary",),
        ),
    )(x)


def _normalize_kernel(x, stats, g2, b2):
    m_per, n_per = x.shape
    n_blocks = m_per // BM

    def body(x_ref, s_ref, g_ref, b_ref, o_ref):
        xb = x_ref[:, :]
        mean_c = s_ref[0, :].reshape(BM, 1)
        rstd_c = s_ref[1, :].reshape(BM, 1)
        o_ref[:, :] = (xb - mean_c) * rstd_c * g_ref[:, :] + b_ref[:, :]

    return pl.pallas_call(
        body,
        grid=(n_blocks,),
        in_specs=[
            pl.BlockSpec((BM, n_per), lambda i: (i, 0)),
            pl.BlockSpec((2, BM), lambda i: (0, i)),
            pl.BlockSpec((1, n_per), lambda i: (0, 0)),
            pl.BlockSpec((1, n_per), lambda i: (0, 0)),
        ],
        out_specs=pl.BlockSpec((BM, n_per), lambda i: (i, 0)),
        out_shape=jax.ShapeDtypeStruct((m_per, n_per), jnp.float32),
        compiler_params=pltpu.CompilerParams(
            dimension_semantics=("arbitrary",),
            vmem_limit_bytes=40 * 1024 * 1024,
        ),
    )(x, stats, g2, b2)


def kernel(x, gamma, beta):
    m_per, n_per = x.shape
    stats = _stats_kernel(x)
    g2 = gamma.reshape(1, n_per)
    b2 = beta.reshape(1, n_per)
    return _normalize_kernel(x, stats, g2, b2)


# device time: 60191 ns/iter; 1.0574x vs baseline; 1.0574x over previous
import jax
import jax.numpy as jnp
from jax import lax
from jax.experimental import pallas as pl
from jax.experimental.pallas import tpu as pltpu

N_GLOBAL = 4096
EPS = 1e-5
BM = 1024


BM_STATS = 2048


def _stats_kernel(x):
    m_per, n_per = x.shape
    n_blocks = m_per // BM_STATS

    def body(x_ref, o_ref, acc, recv, send_sem, recv_sem):
        i = pl.program_id(0)
        my_x = lax.axis_index("x")
        my_y = lax.axis_index("y")

        xb = x_ref[:, :]
        acc[0, pl.ds(i * BM_STATS, BM_STATS)] = jnp.sum(xb, axis=1)
        acc[1, pl.ds(i * BM_STATS, BM_STATS)] = jnp.sum(xb * xb, axis=1)

        @pl.when(i == n_blocks - 1)
        def _():
            rdma = pltpu.make_async_remote_copy(
                src_ref=acc,
                dst_ref=recv,
                send_sem=send_sem,
                recv_sem=recv_sem,
                device_id=(my_x, 1 - my_y),
                device_id_type=pl.DeviceIdType.MESH,
            )
            rdma.start()
            rdma.wait()
            tot_s1 = acc[0, :] + recv[0, :]
            tot_s2 = acc[1, :] + recv[1, :]
            mean = tot_s1 / N_GLOBAL
            var = tot_s2 / N_GLOBAL - mean * mean
            o_ref[0, :] = mean
            o_ref[1, :] = lax.rsqrt(var + EPS)

    return pl.pallas_call(
        body,
        grid=(n_blocks,),
        in_specs=[pl.BlockSpec((BM_STATS, n_per), lambda i: (i, 0))],
        out_specs=pl.BlockSpec((2, m_per), lambda i: (0, 0)),
        out_shape=jax.ShapeDtypeStruct((2, m_per), jnp.float32),
        scratch_shapes=[
            pltpu.VMEM((2, m_per), jnp.float32),
            pltpu.VMEM((2, m_per), jnp.float32),
            pltpu.SemaphoreType.DMA,
            pltpu.SemaphoreType.DMA,
        ],
        compiler_params=pltpu.CompilerParams(
            dimension_semantics=("arbitrary",),
            vmem_limit_bytes=48 * 1024 * 1024,
        ),
    )(x)


def _normalize_kernel(x, stats, g2, b2):
    m_per, n_per = x.shape
    n_blocks = m_per // BM

    def body(x_ref, s_ref, g_ref, b_ref, o_ref):
        xb = x_ref[:, :]
        mean_c = s_ref[0, :].reshape(BM, 1)
        rstd_c = s_ref[1, :].reshape(BM, 1)
        o_ref[:, :] = (xb - mean_c) * rstd_c * g_ref[:, :] + b_ref[:, :]

    return pl.pallas_call(
        body,
        grid=(n_blocks,),
        in_specs=[
            pl.BlockSpec((BM, n_per), lambda i: (i, 0)),
            pl.BlockSpec((2, BM), lambda i: (0, i)),
            pl.BlockSpec((1, n_per), lambda i: (0, 0)),
            pl.BlockSpec((1, n_per), lambda i: (0, 0)),
        ],
        out_specs=pl.BlockSpec((BM, n_per), lambda i: (i, 0)),
        out_shape=jax.ShapeDtypeStruct((m_per, n_per), jnp.float32),
        compiler_params=pltpu.CompilerParams(
            dimension_semantics=("arbitrary",),
            vmem_limit_bytes=40 * 1024 * 1024,
        ),
    )(x, stats, g2, b2)


def kernel(x, gamma, beta):
    m_per, n_per = x.shape
    stats = _stats_kernel(x)
    g2 = gamma.reshape(1, n_per)
    b2 = beta.reshape(1, n_per)
    return _normalize_kernel(x, stats, g2, b2)


# device time: 55274 ns/iter; 1.1514x vs baseline; 1.0890x over previous
import jax
import jax.numpy as jnp
from jax import lax
from jax.experimental import pallas as pl
from jax.experimental.pallas import tpu as pltpu

N_GLOBAL = 4096
EPS = 1e-5
BM = 1024


BM_STATS = 1024


def _stats_kernel(x):
    m_per, n_per = x.shape
    n_blocks = m_per // BM_STATS
    head = (n_blocks - 1) * BM_STATS

    def body(x_ref, o_ref, acc, recv, sem1a, sem1b, sem2a, sem2b):
        i = pl.program_id(0)
        my_x = lax.axis_index("x")
        my_y = lax.axis_index("y")
        nbr = (my_x, 1 - my_y)

        def mk(lo, sz, s_sem, r_sem):
            return pltpu.make_async_remote_copy(
                src_ref=acc.at[:, pl.ds(lo, sz)],
                dst_ref=recv.at[:, pl.ds(lo, sz)],
                send_sem=s_sem,
                recv_sem=r_sem,
                device_id=nbr,
                device_id_type=pl.DeviceIdType.MESH,
            )

        @pl.when(i == 0)
        def _():
            barrier_sem = pltpu.get_barrier_semaphore()
            pl.semaphore_signal(
                barrier_sem, inc=1,
                device_id=nbr, device_id_type=pl.DeviceIdType.MESH,
            )
            pl.semaphore_wait(barrier_sem, 1)

        @pl.when(i == n_blocks - 1)
        def _():
            mk(0, head, sem1a, sem1b).start()

        xb = x_ref[:, :]
        acc[0, pl.ds(i * BM_STATS, BM_STATS)] = jnp.sum(xb, axis=1)
        acc[1, pl.ds(i * BM_STATS, BM_STATS)] = jnp.sum(xb * xb, axis=1)

        @pl.when(i == n_blocks - 1)
        def _():
            mk(head, BM_STATS, sem2a, sem2b).start()
            mk(0, head, sem1a, sem1b).wait()
            mk(head, BM_STATS, sem2a, sem2b).wait()
            tot_s1 = acc[0, :] + recv[0, :]
            tot_s2 = acc[1, :] + recv[1, :]
            mean = tot_s1 / N_GLOBAL
            var = tot_s2 / N_GLOBAL - mean * mean
            o_ref[0, :] = mean
            o_ref[1, :] = lax.rsqrt(var + EPS)

    return pl.pallas_call(
        body,
        grid=(n_blocks,),
        in_specs=[pl.BlockSpec((BM_STATS, n_per), lambda i: (i, 0))],
        out_specs=pl.BlockSpec((2, m_per), lambda i: (0, 0)),
        out_shape=jax.ShapeDtypeStruct((2, m_per), jnp.float32),
        scratch_shapes=[
            pltpu.VMEM((2, m_per), jnp.float32),
            pltpu.VMEM((2, m_per), jnp.float32),
            pltpu.SemaphoreType.DMA,
            pltpu.SemaphoreType.DMA,
            pltpu.SemaphoreType.DMA,
            pltpu.SemaphoreType.DMA,
        ],
        compiler_params=pltpu.CompilerParams(
            dimension_semantics=("arbitrary",),
            vmem_limit_bytes=48 * 1024 * 1024,
            collective_id=0,
        ),
    )(x)


def _normalize_kernel(x, stats, g2, b2):
    m_per, n_per = x.shape
    n_blocks = m_per // BM

    def body(x_ref, s_ref, g_ref, b_ref, o_ref):
        xb = x_ref[:, :]
        mean_c = s_ref[0, :].reshape(BM, 1)
        rstd_c = s_ref[1, :].reshape(BM, 1)
        o_ref[:, :] = (xb - mean_c) * rstd_c * g_ref[:, :] + b_ref[:, :]

    return pl.pallas_call(
        body,
        grid=(n_blocks,),
        in_specs=[
            pl.BlockSpec((BM, n_per), lambda i: (i, 0)),
            pl.BlockSpec((2, BM), lambda i: (0, i)),
            pl.BlockSpec((1, n_per), lambda i: (0, 0)),
            pl.BlockSpec((1, n_per), lambda i: (0, 0)),
        ],
        out_specs=pl.BlockSpec((BM, n_per), lambda i: (i, 0)),
        out_shape=jax.ShapeDtypeStruct((m_per, n_per), jnp.float32),
        compiler_params=pltpu.CompilerParams(
            dimension_semantics=("arbitrary",),
            vmem_limit_bytes=40 * 1024 * 1024,
        ),
    )(x, stats, g2, b2)


def kernel(x, gamma, beta):
    m_per, n_per = x.shape
    stats = _stats_kernel(x)
    g2 = gamma.reshape(1, n_per)
    b2 = beta.reshape(1, n_per)
    return _normalize_kernel(x, stats, g2, b2)


# device time: 54842 ns/iter; 1.1605x vs baseline; 1.0079x over previous
import jax
import jax.numpy as jnp
from jax import lax
from jax.experimental import pallas as pl
from jax.experimental.pallas import tpu as pltpu

N_GLOBAL = 4096
EPS = 1e-5
BM = 1536


BM_STATS = 1024


def _stats_kernel(x):
    m_per, n_per = x.shape
    n_blocks = m_per // BM_STATS
    head = (n_blocks - 1) * BM_STATS

    def body(x_ref, o_ref, acc, recv, sem1a, sem1b, sem2a, sem2b):
        i = pl.program_id(0)
        my_x = lax.axis_index("x")
        my_y = lax.axis_index("y")
        nbr = (my_x, 1 - my_y)

        def mk(lo, sz, s_sem, r_sem):
            return pltpu.make_async_remote_copy(
                src_ref=acc.at[:, pl.ds(lo, sz)],
                dst_ref=recv.at[:, pl.ds(lo, sz)],
                send_sem=s_sem,
                recv_sem=r_sem,
                device_id=nbr,
                device_id_type=pl.DeviceIdType.MESH,
            )

        @pl.when(i == 0)
        def _():
            barrier_sem = pltpu.get_barrier_semaphore()
            pl.semaphore_signal(
                barrier_sem, inc=1,
                device_id=nbr, device_id_type=pl.DeviceIdType.MESH,
            )
            pl.semaphore_wait(barrier_sem, 1)

        @pl.when(i == n_blocks - 1)
        def _():
            mk(0, head, sem1a, sem1b).start()

        xb = x_ref[:, :]
        acc[0, pl.ds(i * BM_STATS, BM_STATS)] = jnp.sum(xb, axis=1)
        acc[1, pl.ds(i * BM_STATS, BM_STATS)] = jnp.sum(xb * xb, axis=1)

        @pl.when(i == n_blocks - 1)
        def _():
            mk(head, BM_STATS, sem2a, sem2b).start()
            mk(0, head, sem1a, sem1b).wait()
            mk(head, BM_STATS, sem2a, sem2b).wait()
            tot_s1 = acc[0, :] + recv[0, :]
            tot_s2 = acc[1, :] + recv[1, :]
            mean = tot_s1 / N_GLOBAL
            var = tot_s2 / N_GLOBAL - mean * mean
            o_ref[0, :] = mean
            o_ref[1, :] = lax.rsqrt(var + EPS)

    return pl.pallas_call(
        body,
        grid=(n_blocks,),
        in_specs=[pl.BlockSpec((BM_STATS, n_per), lambda i: (i, 0))],
        out_specs=pl.BlockSpec((2, m_per), lambda i: (0, 0)),
        out_shape=jax.ShapeDtypeStruct((2, m_per), jnp.float32),
        scratch_shapes=[
            pltpu.VMEM((2, m_per), jnp.float32),
            pltpu.VMEM((2, m_per), jnp.float32),
            pltpu.SemaphoreType.DMA,
            pltpu.SemaphoreType.DMA,
            pltpu.SemaphoreType.DMA,
            pltpu.SemaphoreType.DMA,
        ],
        compiler_params=pltpu.CompilerParams(
            dimension_semantics=("arbitrary",),
            vmem_limit_bytes=48 * 1024 * 1024,
            collective_id=0,
        ),
    )(x)


def _normalize_kernel(x, stats, g2, b2):
    m_per, n_per = x.shape
    n_blocks = m_per // BM

    def body(x_ref, s_ref, g_ref, b_ref, o_ref):
        xb = x_ref[:, :]
        mean_c = s_ref[0, :].reshape(BM, 1)
        rstd_c = s_ref[1, :].reshape(BM, 1)
        o_ref[:, :] = (xb - mean_c) * rstd_c * g_ref[:, :] + b_ref[:, :]

    return pl.pallas_call(
        body,
        grid=(n_blocks,),
        in_specs=[
            pl.BlockSpec((BM, n_per), lambda i: (i, 0)),
            pl.BlockSpec((2, BM), lambda i: (0, i)),
            pl.BlockSpec((1, n_per), lambda i: (0, 0)),
            pl.BlockSpec((1, n_per), lambda i: (0, 0)),
        ],
        out_specs=pl.BlockSpec((BM, n_per), lambda i: (i, 0)),
        out_shape=jax.ShapeDtypeStruct((m_per, n_per), jnp.float32),
        compiler_params=pltpu.CompilerParams(
            dimension_semantics=("arbitrary",),
            vmem_limit_bytes=56 * 1024 * 1024,
        ),
    )(x, stats, g2, b2)


def kernel(x, gamma, beta):
    m_per, n_per = x.shape
    stats = _stats_kernel(x)
    g2 = gamma.reshape(1, n_per)
    b2 = beta.reshape(1, n_per)
    return _normalize_kernel(x, stats, g2, b2)
